# Initial kernel scaffold; baseline (speedup 1.0000x reference)
import jax
import jax.numpy as jnp
from jax.experimental import pallas as pl


def kernel(x, edge_index, W1a, b1a, W1b, b1b, Wc1, bc1, Wc2, bc2, Wl, bl):
    raise NotImplementedError("write your pallas kernel here")


# speedup vs baseline: 2.5740x; 2.5740x over previous
import jax
import jax.numpy as jnp
from jax.experimental import pallas as pl
from jax.experimental.pallas import tpu as pltpu

N_NODES = 100000
ROW_BLK = 1000


def _final_linear_body(h_ref, w_ref, b_ref, o_ref):
    o_ref[...] = h_ref[...] @ w_ref[...] + b_ref[...]


def _final_linear(h, W, b):
    n, f = h.shape
    grid = n // ROW_BLK
    return pl.pallas_call(
        _final_linear_body,
        grid=(grid,),
        in_specs=[
            pl.BlockSpec((ROW_BLK, f), lambda i: (i, 0)),
            pl.BlockSpec((f, W.shape[1]), lambda i: (0, 0)),
            pl.BlockSpec((1, W.shape[1]), lambda i: (0, 0)),
        ],
        out_specs=pl.BlockSpec((ROW_BLK, W.shape[1]), lambda i: (i, 0)),
        out_shape=jax.ShapeDtypeStruct((n, W.shape[1]), h.dtype),
    )(h, W, b.reshape(1, -1))


def kernel(x, edge_index, W1a, b1a, W1b, b1b, Wc1, bc1, Wc2, bc2, Wl, bl):
    n = x.shape[0]
    src = edge_index[0]
    dst = edge_index[1]
    deg = jnp.ones((n,), jnp.float32).at[dst].add(1.0)
    dinv = jax.lax.rsqrt(deg)

    h = jax.nn.elu(x @ W1a + b1a)
    h = jax.nn.elu(h @ W1b + b1b)

    g1 = dinv[:, None] * (h @ Wc1)
    acc1 = jnp.zeros((n, g1.shape[1]), jnp.float32).at[dst].add(g1[src])
    o1 = jax.nn.relu(dinv[:, None] * (acc1 + g1) + bc1)

    g2 = dinv[:, None] * (o1 @ Wc2)
    acc2 = jnp.zeros((n, g2.shape[1]), jnp.float32).at[dst].add(g2[src])
    o2 = dinv[:, None] * (acc2 + g2) + bc2

    return _final_linear(o2, Wl, bl)


# speedup vs baseline: 28.7521x; 11.1703x over previous
import functools

import jax
import jax.numpy as jnp
from jax import lax
from jax.experimental import pallas as pl
from jax.experimental.pallas import tpu as pltpu
from jax.experimental.pallas import tpu_sc as plsc

N = 100000
E = 1600000
NC, NS = 2, 16
PAD_ROWS = 96
NP = N + PAD_ROWS
NSEG = NP // NS
CH = 128
ER = 12544
EP = ER * CH
SLAB = 56
ROW_BLK = 1000

_mesh = plsc.VectorSubcoreMesh(
    core_axis_name="c", subcore_axis_name="s", num_cores=NC, num_subcores=NS)
_sc_params = pltpu.CompilerParams(use_tc_tiling_on_sc=False)


def _edge_slab(table, row0, src_hbm, dst_hbm, src_sb, dst_sb, bufs, sems,
               acc):
    pltpu.sync_copy(src_hbm.at[pl.ds(row0, SLAB), :], src_sb)
    pltpu.sync_copy(dst_hbm.at[pl.ds(row0, SLAB), :], dst_sb)

    def start(j, b):
        pltpu.async_copy(table.at[src_sb.at[j]], bufs[b], sems[b])

    def wait(j, b):
        pltpu.make_async_copy(table.at[src_sb.at[j]], bufs[b], sems[b]).wait()

    start(0, 0)
    start(1, 1)

    def body(t, carry):
        j = t * 2
        for b in range(2):
            jj = j + b
            wait(jj, b)
            pltpu.sync_copy(bufs[b], acc.at[dst_sb.at[jj]], add=True)

            @pl.when(jj + 2 < SLAB)
            def _():
                start(jj + 2, b)
        return carry

    lax.fori_loop(0, SLAB // 2, body, 0)


def _make_conv_kernel(feature_split):
    scratch = [
        pltpu.VMEM((SLAB, CH), jnp.int32),
        pltpu.VMEM((SLAB, CH), jnp.int32),
        pltpu.VMEM((CH, 16), jnp.float32),
        pltpu.VMEM((CH, 16), jnp.float32),
        pltpu.VMEM_SHARED((NP, 16), jnp.float32),
        pltpu.SemaphoreType.DMA,
        pltpu.SemaphoreType.DMA,
    ]
    n_slabs = (ER // NS if feature_split else ER // (NC * NS)) // SLAB

    @functools.partial(pl.kernel,
                       out_type=jax.ShapeDtypeStruct((NC, NP, 16),
                                                     jnp.float32),
                       mesh=_mesh, scratch_types=scratch,
                       compiler_params=_sc_params)
    def conv(g_hbm, src_hbm, dst_hbm, z_hbm, out_hbm,
             src_sb, dst_sb, buf0, buf1, acc, sem0, sem1):
        c = lax.axis_index("c")
        s = lax.axis_index("s")
        bufs, sems = (buf0, buf1), (sem0, sem1)
        table = g_hbm.at[c] if feature_split else g_hbm
        worker_row0 = (s if feature_split else c * NS + s) * (n_slabs * SLAB)
        pltpu.sync_copy(z_hbm.at[pl.ds(s * NSEG, NSEG), :],
                        acc.at[pl.ds(s * NSEG, NSEG), :])
        plsc.subcore_barrier()
        for t in range(n_slabs):
            _edge_slab(table, worker_row0 + t * SLAB,
                       src_hbm, dst_hbm, src_sb, dst_sb, bufs, sems, acc)
        plsc.subcore_barrier()
        pltpu.sync_copy(acc.at[pl.ds(s * NSEG, NSEG), :],
                        out_hbm.at[c, pl.ds(s * NSEG, NSEG), :])

    return conv


_conv1_sc = _make_conv_kernel(True)
_conv2_sc = _make_conv_kernel(False)


@functools.partial(
    pl.kernel,
    out_type=jax.ShapeDtypeStruct((NC, NP), jnp.float32),
    mesh=_mesh,
    scratch_types=[
        pltpu.VMEM((SLAB, CH), jnp.int32),
        pltpu.VMEM((CH,), jnp.float32),
        pltpu.VMEM_SHARED((NP,), jnp.float32),
    ],
    compiler_params=_sc_params,
)
def _deg_sc(dst_hbm, ones_hbm, z_hbm, out_hbm, dst_sb, ones_v, acc):
    c = lax.axis_index("c")
    s = lax.axis_index("s")
    nseg = NP // NS
    n_slabs = ER // (NC * NS) // SLAB
    pltpu.sync_copy(z_hbm.at[pl.ds(s * nseg, nseg)],
                    acc.at[pl.ds(s * nseg, nseg)])
    pltpu.sync_copy(ones_hbm, ones_v)
    plsc.subcore_barrier()
    row0 = (c * NS + s) * (n_slabs * SLAB)
    for t in range(n_slabs):
        pltpu.sync_copy(dst_hbm.at[pl.ds(row0 + t * SLAB, SLAB), :], dst_sb)

        def body(j, carry):
            pltpu.sync_copy(ones_v, acc.at[dst_sb.at[j]], add=True)
            return carry

        lax.fori_loop(0, SLAB, body, 0)
    plsc.subcore_barrier()
    pltpu.sync_copy(acc.at[pl.ds(s * nseg, nseg)],
                    out_hbm.at[c, pl.ds(s * nseg, nseg)])


def _elu(v):
    return jnp.where(v > 0, v, jnp.exp(v) - 1.0)


def _mlp_body(x_ref, w1a_ref, b1a_ref, w1b_ref, b1b_ref, wc1_ref, deg_ref,
              g1_ref, dinv_ref):
    h = _elu(x_ref[...] @ w1a_ref[...] + b1a_ref[...])
    h = _elu(h @ w1b_ref[...] + b1b_ref[...])
    deg = deg_ref[0, :, 0] + deg_ref[1, :, 0] + 1.0
    dinv = lax.rsqrt(deg)[:, None]
    g = dinv * (h @ wc1_ref[...])
    g1_ref[0] = g[:, :16]
    g1_ref[1] = g[:, 16:]
    dinv_ref[...] = dinv


def _mid_body(acc1_ref, g1_ref, dinv_ref, bc1_ref, wc2_ref, g2_ref):
    dinv = dinv_ref[...]
    o1 = jnp.maximum(dinv[None] * (acc1_ref[...] + g1_ref[...]) + bc1_ref[...],
                     0.0)
    o1f = jnp.concatenate([o1[0], o1[1]], axis=1)
    g2_ref[...] = dinv * (o1f @ wc2_ref[...])


def _out_body(acc2_ref, g2_ref, dinv_ref, bc2_ref, wl_ref, bl_ref, z_ref):
    o2 = (dinv_ref[...] * (acc2_ref[0] + acc2_ref[1] + g2_ref[...])
          + bc2_ref[...])
    z_ref[...] = o2 @ wl_ref[...] + bl_ref[...]


def _full(shape):
    return pl.BlockSpec(shape, lambda i: tuple(0 for _ in shape))


def kernel(x, edge_index, W1a, b1a, W1b, b1b, Wc1, bc1, Wc2, bc2, Wl, bl):
    src, dst = edge_index[0], edge_index[1]
    pad = EP - E
    pidx = N + (jnp.arange(pad, dtype=jnp.int32) % PAD_ROWS)
    src2 = jnp.concatenate([src, pidx]).reshape(ER, CH)
    dst2 = jnp.concatenate([dst, pidx]).reshape(ER, CH)
    zeros16 = jnp.zeros((NP, 16), jnp.float32)
    zeros1 = jnp.zeros((NP,), jnp.float32)
    ones_col = jnp.ones((CH,), jnp.float32)

    degp = _deg_sc(dst2, ones_col, zeros1).reshape(NC, NP, 1)

    grid = N // ROW_BLK
    g1, dinv = pl.pallas_call(
        _mlp_body,
        grid=(grid,),
        in_specs=[
            pl.BlockSpec((ROW_BLK, 128), lambda i: (i, 0)),
            _full((128, 64)), _full((1, 64)),
            _full((64, 32)), _full((1, 32)),
            _full((32, 32)),
            pl.BlockSpec((NC, ROW_BLK, 1), lambda i: (0, i, 0)),
        ],
        out_specs=[
            pl.BlockSpec((NC, ROW_BLK, 16), lambda i: (0, i, 0)),
            pl.BlockSpec((ROW_BLK, 1), lambda i: (i, 0)),
        ],
        out_shape=[
            jax.ShapeDtypeStruct((NC, NP, 16), jnp.float32),
            jax.ShapeDtypeStruct((N, 1), jnp.float32),
        ],
    )(x, W1a, b1a.reshape(1, -1), W1b, b1b.reshape(1, -1), Wc1, degp)

    acc1 = _conv1_sc(g1, src2, dst2, zeros16)

    g2 = pl.pallas_call(
        _mid_body,
        grid=(grid,),
        in_specs=[
            pl.BlockSpec((NC, ROW_BLK, 16), lambda i: (0, i, 0)),
            pl.BlockSpec((NC, ROW_BLK, 16), lambda i: (0, i, 0)),
            pl.BlockSpec((ROW_BLK, 1), lambda i: (i, 0)),
            _full((NC, 1, 16)),
            _full((32, 16)),
        ],
        out_specs=pl.BlockSpec((ROW_BLK, 16), lambda i: (i, 0)),
        out_shape=jax.ShapeDtypeStruct((NP, 16), jnp.float32),
    )(acc1, g1, dinv, bc1.reshape(NC, 1, 16), Wc2)

    acc2 = _conv2_sc(g2, src2, dst2, zeros16)

    z = pl.pallas_call(
        _out_body,
        grid=(grid,),
        in_specs=[
            pl.BlockSpec((NC, ROW_BLK, 16), lambda i: (0, i, 0)),
            pl.BlockSpec((ROW_BLK, 16), lambda i: (i, 0)),
            pl.BlockSpec((ROW_BLK, 1), lambda i: (i, 0)),
            _full((1, 16)),
            _full((16, 16)), _full((1, 16)),
        ],
        out_specs=pl.BlockSpec((ROW_BLK, 16), lambda i: (i, 0)),
        out_shape=jax.ShapeDtypeStruct((N, 16), jnp.float32),
    )(acc2, g2, dinv, bc2.reshape(1, -1), Wl, bl.reshape(1, -1))

    return z


# speedup vs baseline: 33.1051x; 1.1514x over previous
import functools

import jax
import jax.numpy as jnp
from jax import lax
from jax.experimental import pallas as pl
from jax.experimental.pallas import tpu as pltpu
from jax.experimental.pallas import tpu_sc as plsc

N = 100000
E = 1600000
NC, NS = 2, 16
PAD_ROWS = 96
NP = N + PAD_ROWS
NSEG = NP // NS
CH = 128
ER = 12544
EP = ER * CH
SLAB = 56
ROW_BLK = 1000

_mesh = plsc.VectorSubcoreMesh(
    core_axis_name="c", subcore_axis_name="s", num_cores=NC, num_subcores=NS)
_sc_params = pltpu.CompilerParams(use_tc_tiling_on_sc=False)


NBUF = 4


def _edge_slab(table, row0, src_hbm, dst_hbm, src_sb, dst_sb, bufs, gsems,
               ssems, acc):
    pltpu.sync_copy(src_hbm.at[pl.ds(row0, SLAB), :], src_sb)
    pltpu.sync_copy(dst_hbm.at[pl.ds(row0, SLAB), :], dst_sb)

    def start_g(j, b):
        pltpu.async_copy(table.at[src_sb.at[j]], bufs[b], gsems[b])

    def wait_g(j, b):
        pltpu.make_async_copy(table.at[src_sb.at[j]], bufs[b],
                              gsems[b]).wait()

    def start_s(j, b):
        pltpu.async_copy(bufs[b], acc.at[dst_sb.at[j]], ssems[b], add=True)

    def wait_s(j, b):
        pltpu.make_async_copy(bufs[b], acc.at[dst_sb.at[j]],
                              ssems[b]).wait()

    for b in range(NBUF):
        start_g(b, b)

    def body(t, carry):
        j = t * NBUF
        for b in range(NBUF):
            wait_g(j + b, b)
            start_s(j + b, b)
        for b in range(NBUF):
            jn = j + b + NBUF

            @pl.when(jn < SLAB)
            def _():
                wait_s(j + b, b)
                start_g(jn, b)
        return carry

    lax.fori_loop(0, SLAB // NBUF, body, 0)
    for b in range(NBUF):
        wait_s(SLAB - NBUF + b, b)


def _make_conv_kernel(feature_split):
    scratch = (
        [pltpu.VMEM((SLAB, CH), jnp.int32)] * 2
        + [pltpu.VMEM((CH, 16), jnp.float32)] * NBUF
        + [pltpu.VMEM_SHARED((NP, 16), jnp.float32)]
        + [pltpu.SemaphoreType.DMA] * (2 * NBUF)
    )
    n_slabs = (ER // NS if feature_split else ER // (NC * NS)) // SLAB

    @functools.partial(pl.kernel,
                       out_type=jax.ShapeDtypeStruct((NC, NP, 16),
                                                     jnp.float32),
                       mesh=_mesh, scratch_types=scratch,
                       compiler_params=_sc_params)
    def conv(g_hbm, src_hbm, dst_hbm, z_hbm, out_hbm,
             src_sb, dst_sb, *rest):
        bufs = rest[:NBUF]
        acc = rest[NBUF]
        gsems = rest[NBUF + 1:NBUF + 1 + NBUF]
        ssems = rest[NBUF + 1 + NBUF:]
        c = lax.axis_index("c")
        s = lax.axis_index("s")
        table = g_hbm.at[c] if feature_split else g_hbm
        worker_row0 = (s if feature_split else c * NS + s) * (n_slabs * SLAB)
        pltpu.sync_copy(z_hbm.at[pl.ds(s * NSEG, NSEG), :],
                        acc.at[pl.ds(s * NSEG, NSEG), :])
        plsc.subcore_barrier()
        for t in range(n_slabs):
            _edge_slab(table, worker_row0 + t * SLAB,
                       src_hbm, dst_hbm, src_sb, dst_sb, bufs, gsems, ssems,
                       acc)
        plsc.subcore_barrier()
        pltpu.sync_copy(acc.at[pl.ds(s * NSEG, NSEG), :],
                        out_hbm.at[c, pl.ds(s * NSEG, NSEG), :])

    return conv


_conv1_sc = _make_conv_kernel(True)
_conv2_sc = _make_conv_kernel(False)


@functools.partial(
    pl.kernel,
    out_type=jax.ShapeDtypeStruct((NC, NP), jnp.float32),
    mesh=_mesh,
    scratch_types=[
        pltpu.VMEM((SLAB, CH), jnp.int32),
        pltpu.VMEM((CH,), jnp.float32),
        pltpu.VMEM_SHARED((NP,), jnp.float32),
    ],
    compiler_params=_sc_params,
)
def _deg_sc(dst_hbm, ones_hbm, z_hbm, out_hbm, dst_sb, ones_v, acc):
    c = lax.axis_index("c")
    s = lax.axis_index("s")
    nseg = NP // NS
    n_slabs = ER // (NC * NS) // SLAB
    pltpu.sync_copy(z_hbm.at[pl.ds(s * nseg, nseg)],
                    acc.at[pl.ds(s * nseg, nseg)])
    pltpu.sync_copy(ones_hbm, ones_v)
    plsc.subcore_barrier()
    row0 = (c * NS + s) * (n_slabs * SLAB)
    for t in range(n_slabs):
        pltpu.sync_copy(dst_hbm.at[pl.ds(row0 + t * SLAB, SLAB), :], dst_sb)

        def body(j, carry):
            pltpu.sync_copy(ones_v, acc.at[dst_sb.at[j]], add=True)
            return carry

        lax.fori_loop(0, SLAB, body, 0)
    plsc.subcore_barrier()
    pltpu.sync_copy(acc.at[pl.ds(s * nseg, nseg)],
                    out_hbm.at[c, pl.ds(s * nseg, nseg)])


def _elu(v):
    return jnp.where(v > 0, v, jnp.exp(v) - 1.0)


def _mlp_body(x_ref, w1a_ref, b1a_ref, w1b_ref, b1b_ref, wc1_ref, g1_ref):
    h = _elu(x_ref[...] @ w1a_ref[...] + b1a_ref[...])
    h = _elu(h @ w1b_ref[...] + b1b_ref[...])
    g = h @ wc1_ref[...]
    g1_ref[0] = g[:, :16]
    g1_ref[1] = g[:, 16:]


def _mid_body(o1p_ref, bc1_ref, wc2_ref, g2_ref):
    o1 = jnp.maximum(o1p_ref[...] + bc1_ref[...], 0.0)
    o1f = jnp.concatenate([o1[0], o1[1]], axis=1)
    g2_ref[...] = o1f @ wc2_ref[...]


def _out_body(o2p_ref, bc2_ref, wl_ref, bl_ref, z_ref):
    z_ref[...] = (o2p_ref[...] + bc2_ref[...]) @ wl_ref[...] + bl_ref[...]


def _full(shape):
    return pl.BlockSpec(shape, lambda i: tuple(0 for _ in shape))


def kernel(x, edge_index, W1a, b1a, W1b, b1b, Wc1, bc1, Wc2, bc2, Wl, bl):
    src, dst = edge_index[0], edge_index[1]
    pad = EP - E
    pidx = N + (jnp.arange(pad, dtype=jnp.int32) % PAD_ROWS)
    src2 = jnp.concatenate([src, pidx]).reshape(ER, CH)
    dst2 = jnp.concatenate([dst, pidx]).reshape(ER, CH)
    zeros16 = jnp.zeros((NP, 16), jnp.float32)
    zeros1 = jnp.zeros((NP,), jnp.float32)
    ones_col = jnp.ones((CH,), jnp.float32)

    degp = _deg_sc(dst2, ones_col, zeros1)
    dinv = lax.rsqrt(degp[0] + degp[1] + 1.0)
    dcol = dinv[:, None]

    grid = N // ROW_BLK
    hw1 = pl.pallas_call(
        _mlp_body,
        grid=(grid,),
        in_specs=[
            pl.BlockSpec((ROW_BLK, 128), lambda i: (i, 0)),
            _full((128, 64)), _full((1, 64)),
            _full((64, 32)), _full((1, 32)),
            _full((32, 32)),
        ],
        out_specs=pl.BlockSpec((NC, ROW_BLK, 16), lambda i: (0, i, 0)),
        out_shape=jax.ShapeDtypeStruct((NC, NP, 16), jnp.float32),
    )(x, W1a, b1a.reshape(1, -1), W1b, b1b.reshape(1, -1), Wc1)

    g1 = hw1 * dcol[None]
    acc1 = _conv1_sc(g1, src2, dst2, zeros16)
    o1pre = (acc1 + g1) * dcol[None]

    g2u = pl.pallas_call(
        _mid_body,
        grid=(grid,),
        in_specs=[
            pl.BlockSpec((NC, ROW_BLK, 16), lambda i: (0, i, 0)),
            _full((NC, 1, 16)),
            _full((32, 16)),
        ],
        out_specs=pl.BlockSpec((ROW_BLK, 16), lambda i: (i, 0)),
        out_shape=jax.ShapeDtypeStruct((NP, 16), jnp.float32),
    )(o1pre, bc1.reshape(NC, 1, 16), Wc2)

    g2 = g2u * dcol
    acc2 = _conv2_sc(g2, src2, dst2, zeros16)
    o2pre = (acc2[0] + acc2[1] + g2) * dcol

    z = pl.pallas_call(
        _out_body,
        grid=(grid,),
        in_specs=[
            pl.BlockSpec((ROW_BLK, 16), lambda i: (i, 0)),
            _full((1, 16)),
            _full((16, 16)), _full((1, 16)),
        ],
        out_specs=pl.BlockSpec((ROW_BLK, 16), lambda i: (i, 0)),
        out_shape=jax.ShapeDtypeStruct((N, 16), jnp.float32),
    )(o2pre, bc2.reshape(1, -1), Wl, bl.reshape(1, -1))

    return z


# speedup vs baseline: 36.4789x; 1.1019x over previous
import functools

import jax
import jax.numpy as jnp
from jax import lax
from jax.experimental import pallas as pl
from jax.experimental.pallas import tpu as pltpu
from jax.experimental.pallas import tpu_sc as plsc

N = 100000
E = 1600000
NC, NS = 2, 16
PAD_ROWS = 96
NP = N + PAD_ROWS
NSEG = NP // NS
CH = 128
ER = 12544
EP = ER * CH
SLAB = 56
ROW_BLK = 1000

_mesh = plsc.VectorSubcoreMesh(
    core_axis_name="c", subcore_axis_name="s", num_cores=NC, num_subcores=NS)
_sc_params = pltpu.CompilerParams(use_tc_tiling_on_sc=False)


NBUF = 4


def _edge_slab(table, row0, src_hbm, dst_hbm, src_sb, dst_sb, bufs, gsems,
               ssems, acc):
    pltpu.sync_copy(src_hbm.at[pl.ds(row0, SLAB), :], src_sb)
    pltpu.sync_copy(dst_hbm.at[pl.ds(row0, SLAB), :], dst_sb)

    def start_g(j, b):
        pltpu.async_copy(table.at[src_sb.at[j]], bufs[b], gsems[b])

    def wait_g(j, b):
        pltpu.make_async_copy(table.at[src_sb.at[j]], bufs[b],
                              gsems[b]).wait()

    def start_s(j, b):
        pltpu.async_copy(bufs[b], acc.at[dst_sb.at[j]], ssems[b], add=True)

    def wait_s(j, b):
        pltpu.make_async_copy(bufs[b], acc.at[dst_sb.at[j]],
                              ssems[b]).wait()

    for b in range(NBUF):
        start_g(b, b)

    def body(t, carry):
        j = t * NBUF
        for b in range(NBUF):
            wait_g(j + b, b)
            start_s(j + b, b)
        for b in range(NBUF):
            jn = j + b + NBUF

            @pl.when(jn < SLAB)
            def _():
                wait_s(j + b, b)
                start_g(jn, b)
        return carry

    lax.fori_loop(0, SLAB // NBUF, body, 0)
    for b in range(NBUF):
        wait_s(SLAB - NBUF + b, b)


def _make_conv_kernel(feature_split):
    scratch = (
        [pltpu.VMEM((SLAB, CH), jnp.int32)] * 2
        + [pltpu.VMEM((CH, 16), jnp.float32)] * NBUF
        + [pltpu.VMEM_SHARED((NP, 16), jnp.float32)]
        + [pltpu.SemaphoreType.DMA] * (2 * NBUF)
    )
    n_slabs = (ER // NS if feature_split else ER // (NC * NS)) // SLAB

    @functools.partial(pl.kernel,
                       out_type=jax.ShapeDtypeStruct((NC, NP, 16),
                                                     jnp.float32),
                       mesh=_mesh, scratch_types=scratch,
                       compiler_params=_sc_params)
    def conv(g_hbm, src_hbm, dst_hbm, z_hbm, out_hbm,
             src_sb, dst_sb, *rest):
        bufs = rest[:NBUF]
        acc = rest[NBUF]
        gsems = rest[NBUF + 1:NBUF + 1 + NBUF]
        ssems = rest[NBUF + 1 + NBUF:]
        c = lax.axis_index("c")
        s = lax.axis_index("s")
        table = g_hbm.at[c] if feature_split else g_hbm
        worker_row0 = (s if feature_split else c * NS + s) * (n_slabs * SLAB)
        pltpu.sync_copy(z_hbm.at[pl.ds(s * NSEG, NSEG), :],
                        acc.at[pl.ds(s * NSEG, NSEG), :])
        plsc.subcore_barrier()
        for t in range(n_slabs):
            _edge_slab(table, worker_row0 + t * SLAB,
                       src_hbm, dst_hbm, src_sb, dst_sb, bufs, gsems, ssems,
                       acc)
        plsc.subcore_barrier()
        pltpu.sync_copy(acc.at[pl.ds(s * NSEG, NSEG), :],
                        out_hbm.at[c, pl.ds(s * NSEG, NSEG), :])

    return conv


_conv1_sc = _make_conv_kernel(True)
_conv2_sc = _make_conv_kernel(False)


@functools.partial(
    pl.kernel,
    out_type=jax.ShapeDtypeStruct((NC, NP), jnp.float32),
    mesh=_mesh,
    scratch_types=[
        pltpu.VMEM((SLAB, CH), jnp.int32),
        pltpu.VMEM((CH,), jnp.float32),
        pltpu.VMEM_SHARED((NP,), jnp.float32),
    ],
    compiler_params=_sc_params,
)
def _deg_sc(dst_hbm, ones_hbm, z_hbm, out_hbm, dst_sb, ones_v, acc):
    c = lax.axis_index("c")
    s = lax.axis_index("s")
    nseg = NP // NS
    n_slabs = ER // (NC * NS) // SLAB
    pltpu.sync_copy(z_hbm.at[pl.ds(s * nseg, nseg)],
                    acc.at[pl.ds(s * nseg, nseg)])
    pltpu.sync_copy(ones_hbm, ones_v)
    plsc.subcore_barrier()
    row0 = (c * NS + s) * (n_slabs * SLAB)
    for t in range(n_slabs):
        pltpu.sync_copy(dst_hbm.at[pl.ds(row0 + t * SLAB, SLAB), :], dst_sb)

        def body(j, carry):
            pltpu.sync_copy(ones_v, acc.at[dst_sb.at[j]], add=True)
            return carry

        lax.fori_loop(0, SLAB, body, 0)
    plsc.subcore_barrier()
    pltpu.sync_copy(acc.at[pl.ds(s * nseg, nseg)],
                    out_hbm.at[c, pl.ds(s * nseg, nseg)])


def _elu(v):
    return jnp.where(v > 0, v, jnp.exp(v) - 1.0)


def _mlp_body(x_ref, w1a_ref, b1a_ref, w1b_ref, b1b_ref, wc1_ref, dinv_ref,
              g1_ref):
    h = _elu(x_ref[...] @ w1a_ref[...] + b1a_ref[...])
    h = _elu(h @ w1b_ref[...] + b1b_ref[...])
    g = dinv_ref[...] * (h @ wc1_ref[...])
    g1_ref[0] = g[:, :16]
    g1_ref[1] = g[:, 16:]


def _mid_body(acc1_ref, g1_ref, dinv_ref, bc1_ref, wc2_ref, g2_ref):
    dinv = dinv_ref[...]
    o1 = jnp.maximum(dinv[None] * (acc1_ref[...] + g1_ref[...]) + bc1_ref[...],
                     0.0)
    o1f = jnp.concatenate([o1[0], o1[1]], axis=1)
    g2_ref[...] = dinv * (o1f @ wc2_ref[...])


def _out_body(acc2_ref, g2_ref, dinv_ref, bc2_ref, wl_ref, bl_ref, z_ref):
    o2 = (dinv_ref[...] * (acc2_ref[0] + acc2_ref[1] + g2_ref[...])
          + bc2_ref[...])
    z_ref[...] = o2 @ wl_ref[...] + bl_ref[...]


def _full(shape):
    return pl.BlockSpec(shape, lambda i: tuple(0 for _ in shape))


def kernel(x, edge_index, W1a, b1a, W1b, b1b, Wc1, bc1, Wc2, bc2, Wl, bl):
    src, dst = edge_index[0], edge_index[1]
    pad = EP - E
    pidx = N + (jnp.arange(pad, dtype=jnp.int32) % PAD_ROWS)
    src2 = jnp.concatenate([src, pidx]).reshape(ER, CH)
    dst2 = jnp.concatenate([dst, pidx]).reshape(ER, CH)
    zeros16 = jnp.zeros((NP, 16), jnp.float32)
    zeros1 = jnp.zeros((NP,), jnp.float32)
    ones_col = jnp.ones((CH,), jnp.float32)

    degp = _deg_sc(dst2, ones_col, zeros1)
    dinv = lax.rsqrt(degp[0, :N] + degp[1, :N] + 1.0)[:, None]

    grid = N // ROW_BLK
    g1 = pl.pallas_call(
        _mlp_body,
        grid=(grid,),
        in_specs=[
            pl.BlockSpec((ROW_BLK, 128), lambda i: (i, 0)),
            _full((128, 64)), _full((1, 64)),
            _full((64, 32)), _full((1, 32)),
            _full((32, 32)),
            pl.BlockSpec((ROW_BLK, 1), lambda i: (i, 0)),
        ],
        out_specs=pl.BlockSpec((NC, ROW_BLK, 16), lambda i: (0, i, 0)),
        out_shape=jax.ShapeDtypeStruct((NC, NP, 16), jnp.float32),
    )(x, W1a, b1a.reshape(1, -1), W1b, b1b.reshape(1, -1), Wc1, dinv)

    acc1 = _conv1_sc(g1, src2, dst2, zeros16)

    g2 = pl.pallas_call(
        _mid_body,
        grid=(grid,),
        in_specs=[
            pl.BlockSpec((NC, ROW_BLK, 16), lambda i: (0, i, 0)),
            pl.BlockSpec((NC, ROW_BLK, 16), lambda i: (0, i, 0)),
            pl.BlockSpec((ROW_BLK, 1), lambda i: (i, 0)),
            _full((NC, 1, 16)),
            _full((32, 16)),
        ],
        out_specs=pl.BlockSpec((ROW_BLK, 16), lambda i: (i, 0)),
        out_shape=jax.ShapeDtypeStruct((NP, 16), jnp.float32),
    )(acc1, g1, dinv, bc1.reshape(NC, 1, 16), Wc2)

    acc2 = _conv2_sc(g2, src2, dst2, zeros16)

    z = pl.pallas_call(
        _out_body,
        grid=(grid,),
        in_specs=[
            pl.BlockSpec((NC, ROW_BLK, 16), lambda i: (0, i, 0)),
            pl.BlockSpec((ROW_BLK, 16), lambda i: (i, 0)),
            pl.BlockSpec((ROW_BLK, 1), lambda i: (i, 0)),
            _full((1, 16)),
            _full((16, 16)), _full((1, 16)),
        ],
        out_specs=pl.BlockSpec((ROW_BLK, 16), lambda i: (i, 0)),
        out_shape=jax.ShapeDtypeStruct((N, 16), jnp.float32),
    )(acc2, g2, dinv, bc2.reshape(1, -1), Wl, bl.reshape(1, -1))

    return z


# speedup vs baseline: 43.4648x; 1.1915x over previous
import functools

import jax
import jax.numpy as jnp
from jax import lax
from jax.experimental import pallas as pl
from jax.experimental.pallas import tpu as pltpu
from jax.experimental.pallas import tpu_sc as plsc

N = 100000
E = 1600000
NC, NS = 2, 16
PAD_ROWS = 96
NP = N + PAD_ROWS
NSEG = NP // NS
CH = 128
ER = 12544
EP = ER * CH
SLAB = 56
ROW_BLK = 2000

_mesh = plsc.VectorSubcoreMesh(
    core_axis_name="c", subcore_axis_name="s", num_cores=NC, num_subcores=NS)
_sc_params = pltpu.CompilerParams(use_tc_tiling_on_sc=False)


NBUF = 7


def _edge_slab(table, row0, src_hbm, dst_hbm, src_sb, dst_sb, bufs, gsems,
               ssems, acc):
    pltpu.sync_copy(src_hbm.at[pl.ds(row0, SLAB), :], src_sb)
    pltpu.sync_copy(dst_hbm.at[pl.ds(row0, SLAB), :], dst_sb)

    def start_g(j, b):
        pltpu.async_copy(table.at[src_sb.at[j]], bufs[b], gsems[b])

    def wait_g(j, b):
        pltpu.make_async_copy(table.at[src_sb.at[j]], bufs[b],
                              gsems[b]).wait()

    def start_s(j, b):
        pltpu.async_copy(bufs[b], acc.at[dst_sb.at[j]], ssems[b], add=True)

    def wait_s(j, b):
        pltpu.make_async_copy(bufs[b], acc.at[dst_sb.at[j]],
                              ssems[b]).wait()

    for b in range(NBUF):
        start_g(b, b)

    def body(t, carry):
        j = t * NBUF
        for b in range(NBUF):
            wait_g(j + b, b)
            start_s(j + b, b)
        for b in range(NBUF):
            jn = j + b + NBUF

            @pl.when(jn < SLAB)
            def _():
                wait_s(j + b, b)
                start_g(jn, b)
        return carry

    lax.fori_loop(0, SLAB // NBUF, body, 0)
    for b in range(NBUF):
        wait_s(SLAB - NBUF + b, b)


def _make_conv_kernel(feature_split):
    scratch = (
        [pltpu.VMEM((SLAB, CH), jnp.int32)] * 2
        + [pltpu.VMEM((CH, 16), jnp.float32)] * NBUF
        + [pltpu.VMEM_SHARED((NP, 16), jnp.float32)]
        + [pltpu.SemaphoreType.DMA] * (2 * NBUF)
    )
    n_slabs = (ER // NS if feature_split else ER // (NC * NS)) // SLAB

    @functools.partial(pl.kernel,
                       out_type=jax.ShapeDtypeStruct((NC, NP, 16),
                                                     jnp.float32),
                       mesh=_mesh, scratch_types=scratch,
                       compiler_params=_sc_params)
    def conv(g_hbm, src_hbm, dst_hbm, z_hbm, out_hbm,
             src_sb, dst_sb, *rest):
        bufs = rest[:NBUF]
        acc = rest[NBUF]
        gsems = rest[NBUF + 1:NBUF + 1 + NBUF]
        ssems = rest[NBUF + 1 + NBUF:]
        c = lax.axis_index("c")
        s = lax.axis_index("s")
        table = g_hbm.at[c] if feature_split else g_hbm
        worker_row0 = (s if feature_split else c * NS + s) * (n_slabs * SLAB)
        pltpu.sync_copy(z_hbm.at[pl.ds(s * NSEG, NSEG), :],
                        acc.at[pl.ds(s * NSEG, NSEG), :])
        plsc.subcore_barrier()
        for t in range(n_slabs):
            _edge_slab(table, worker_row0 + t * SLAB,
                       src_hbm, dst_hbm, src_sb, dst_sb, bufs, gsems, ssems,
                       acc)
        plsc.subcore_barrier()
        pltpu.sync_copy(acc.at[pl.ds(s * NSEG, NSEG), :],
                        out_hbm.at[c, pl.ds(s * NSEG, NSEG), :])

    return conv


_conv1_sc = _make_conv_kernel(True)
_conv2_sc = _make_conv_kernel(False)


@functools.partial(
    pl.kernel,
    out_type=jax.ShapeDtypeStruct((NC, NP), jnp.float32),
    mesh=_mesh,
    scratch_types=[
        pltpu.VMEM((SLAB, CH), jnp.int32),
        pltpu.VMEM((CH,), jnp.float32),
        pltpu.VMEM_SHARED((NP,), jnp.float32),
    ],
    compiler_params=_sc_params,
)
def _deg_sc(dst_hbm, ones_hbm, z_hbm, out_hbm, dst_sb, ones_v, acc):
    c = lax.axis_index("c")
    s = lax.axis_index("s")
    nseg = NP // NS
    n_slabs = ER // (NC * NS) // SLAB
    pltpu.sync_copy(z_hbm.at[pl.ds(s * nseg, nseg)],
                    acc.at[pl.ds(s * nseg, nseg)])
    pltpu.sync_copy(ones_hbm, ones_v)
    plsc.subcore_barrier()
    row0 = (c * NS + s) * (n_slabs * SLAB)
    for t in range(n_slabs):
        pltpu.sync_copy(dst_hbm.at[pl.ds(row0 + t * SLAB, SLAB), :], dst_sb)

        def body(j, carry):
            pltpu.sync_copy(ones_v, acc.at[dst_sb.at[j]], add=True)
            return carry

        lax.fori_loop(0, SLAB, body, 0)
    plsc.subcore_barrier()
    pltpu.sync_copy(acc.at[pl.ds(s * nseg, nseg)],
                    out_hbm.at[c, pl.ds(s * nseg, nseg)])


def _elu(v):
    return jnp.where(v > 0, v, jnp.exp(v) - 1.0)


def _mlp_body(x_ref, w1a_ref, b1a_ref, w1b_ref, b1b_ref, wc1_ref, dinv_ref,
              g1_ref):
    h = _elu(x_ref[...] @ w1a_ref[...] + b1a_ref[...])
    h = _elu(h @ w1b_ref[...] + b1b_ref[...])
    g = dinv_ref[...] * (h @ wc1_ref[...])
    g1_ref[0] = g[:, :16]
    g1_ref[1] = g[:, 16:]


def _mid_body(acc1_ref, g1_ref, dinv_ref, bc1_ref, wc2_ref, g2_ref):
    dinv = dinv_ref[...]
    o1 = jnp.maximum(dinv[None] * (acc1_ref[...] + g1_ref[...]) + bc1_ref[...],
                     0.0)
    o1f = jnp.concatenate([o1[0], o1[1]], axis=1)
    g2_ref[...] = dinv * (o1f @ wc2_ref[...])


def _out_body(acc2_ref, g2_ref, dinv_ref, bc2_ref, wl_ref, bl_ref, z_ref):
    o2 = (dinv_ref[...] * (acc2_ref[0] + acc2_ref[1] + g2_ref[...])
          + bc2_ref[...])
    z_ref[...] = o2 @ wl_ref[...] + bl_ref[...]


def _full(shape):
    return pl.BlockSpec(shape, lambda i: tuple(0 for _ in shape))


def kernel(x, edge_index, W1a, b1a, W1b, b1b, Wc1, bc1, Wc2, bc2, Wl, bl):
    src, dst = edge_index[0], edge_index[1]
    pad = EP - E
    pidx = N + (jnp.arange(pad, dtype=jnp.int32) % PAD_ROWS)
    src2 = jnp.concatenate([src, pidx]).reshape(ER, CH)
    dst2 = jnp.concatenate([dst, pidx]).reshape(ER, CH)
    zeros16 = jnp.zeros((NP, 16), jnp.float32)
    zeros1 = jnp.zeros((NP,), jnp.float32)
    ones_col = jnp.ones((CH,), jnp.float32)

    degp = _deg_sc(dst2, ones_col, zeros1)
    dinv = lax.rsqrt(degp[0, :N] + degp[1, :N] + 1.0)[:, None]

    grid = N // ROW_BLK
    g1 = pl.pallas_call(
        _mlp_body,
        grid=(grid,),
        in_specs=[
            pl.BlockSpec((ROW_BLK, 128), lambda i: (i, 0)),
            _full((128, 64)), _full((1, 64)),
            _full((64, 32)), _full((1, 32)),
            _full((32, 32)),
            pl.BlockSpec((ROW_BLK, 1), lambda i: (i, 0)),
        ],
        out_specs=pl.BlockSpec((NC, ROW_BLK, 16), lambda i: (0, i, 0)),
        out_shape=jax.ShapeDtypeStruct((NC, NP, 16), jnp.float32),
    )(x, W1a, b1a.reshape(1, -1), W1b, b1b.reshape(1, -1), Wc1, dinv)

    acc1 = _conv1_sc(g1, src2, dst2, zeros16)

    g2 = pl.pallas_call(
        _mid_body,
        grid=(grid,),
        in_specs=[
            pl.BlockSpec((NC, ROW_BLK, 16), lambda i: (0, i, 0)),
            pl.BlockSpec((NC, ROW_BLK, 16), lambda i: (0, i, 0)),
            pl.BlockSpec((ROW_BLK, 1), lambda i: (i, 0)),
            _full((NC, 1, 16)),
            _full((32, 16)),
        ],
        out_specs=pl.BlockSpec((ROW_BLK, 16), lambda i: (i, 0)),
        out_shape=jax.ShapeDtypeStruct((NP, 16), jnp.float32),
    )(acc1, g1, dinv, bc1.reshape(NC, 1, 16), Wc2)

    acc2 = _conv2_sc(g2, src2, dst2, zeros16)

    z = pl.pallas_call(
        _out_body,
        grid=(grid,),
        in_specs=[
            pl.BlockSpec((NC, ROW_BLK, 16), lambda i: (0, i, 0)),
            pl.BlockSpec((ROW_BLK, 16), lambda i: (i, 0)),
            pl.BlockSpec((ROW_BLK, 1), lambda i: (i, 0)),
            _full((1, 16)),
            _full((16, 16)), _full((1, 16)),
        ],
        out_specs=pl.BlockSpec((ROW_BLK, 16), lambda i: (i, 0)),
        out_shape=jax.ShapeDtypeStruct((N, 16), jnp.float32),
    )(acc2, g2, dinv, bc2.reshape(1, -1), Wl, bl.reshape(1, -1))

    return z
